# Initial kernel scaffold; baseline (speedup 1.0000x reference)
import functools

import jax
import jax.numpy as jnp
from jax import lax
from jax.experimental import pallas as pl
from jax.experimental.pallas import tpu as pltpu
from jax.experimental.pallas import tpu_sc as plsc

N = 10000
E = 320000
D = 128

NC = 2
NS = 16
E_PER_SC = E // NC
E_PER_W = E_PER_SC // NS
ROWS_PER_TILE = N // NS

SEG_CHUNK = 400
SEG_ITERS = E_PER_W // SEG_CHUNK

HIST_CHUNK = 2000
HIST_ITERS = E_PER_W // HIST_CHUNK
HW = 16

_sc_mesh = plsc.VectorSubcoreMesh(core_axis_name="c", subcore_axis_name="s")


@functools.partial(
    pl.kernel,
    out_type=jax.ShapeDtypeStruct((NC, N, HW), jnp.float32),
    mesh=_sc_mesh,
    scratch_types=[
        pltpu.VMEM((HIST_CHUNK,), jnp.int32),
        pltpu.VMEM((HIST_CHUNK, HW), jnp.float32),
        pltpu.VMEM((ROWS_PER_TILE, HW), jnp.float32),
        pltpu.VMEM_SHARED((N, HW), jnp.float32),
    ],
)
def _sc_hist(dst_hbm, out_hbm, dst_v, ones_v, zero_v, acc):
    c = lax.axis_index("c")
    s = lax.axis_index("s")

    one16 = jnp.ones((16,), jnp.float32)
    zer16 = jnp.zeros((16,), jnp.float32)

    @pl.loop(0, HIST_CHUNK)
    def _(i):
        ones_v[i, :] = one16

    @pl.loop(0, ROWS_PER_TILE)
    def _(i):
        zero_v[i, :] = zer16

    pltpu.sync_copy(zero_v, acc.at[pl.ds(s * ROWS_PER_TILE, ROWS_PER_TILE)])
    plsc.subcore_barrier()

    base = c * E_PER_SC + s * E_PER_W

    @pl.loop(0, HIST_ITERS)
    def _(i):
        pltpu.sync_copy(dst_hbm.at[pl.ds(base + i * HIST_CHUNK, HIST_CHUNK)],
                        dst_v)
        pltpu.sync_copy(ones_v, acc.at[dst_v], add=True)

    plsc.subcore_barrier()
    row0 = s * ROWS_PER_TILE
    pltpu.sync_copy(acc.at[pl.ds(row0, ROWS_PER_TILE)],
                    out_hbm.at[c, pl.ds(row0, ROWS_PER_TILE)])


@functools.partial(
    pl.kernel,
    out_type=jax.ShapeDtypeStruct((NC, N, D), jnp.float32),
    mesh=_sc_mesh,
    scratch_types=[
        pltpu.VMEM((SEG_CHUNK,), jnp.int32),
        pltpu.VMEM((SEG_CHUNK,), jnp.int32),
        pltpu.VMEM((SEG_CHUNK, D), jnp.float32),
        pltpu.VMEM((ROWS_PER_TILE // 5, D), jnp.float32),
        pltpu.VMEM_SHARED((N, D), jnp.float32),
        pltpu.SemaphoreType.DMA,
    ],
)
def _sc_seg(hs_hbm, src_hbm, dst_hbm, out_hbm,
            src_v, dst_v, rows_v, zero_v, acc, gsem):
    c = lax.axis_index("c")
    s = lax.axis_index("s")

    zer16 = jnp.zeros((16,), jnp.float32)
    zrows = ROWS_PER_TILE // 5

    @pl.loop(0, zrows)
    def _(i):
        for j in range(D // 16):
            zero_v[i, pl.ds(j * 16, 16)] = zer16

    for k in range(5):
        pltpu.sync_copy(
            zero_v, acc.at[pl.ds(s * ROWS_PER_TILE + k * zrows, zrows)])
    plsc.subcore_barrier()

    base = c * E_PER_SC + s * E_PER_W

    @pl.loop(0, SEG_ITERS)
    def _(i):
        e0 = base + i * SEG_CHUNK
        pltpu.sync_copy(src_hbm.at[pl.ds(e0, SEG_CHUNK)], src_v)
        pltpu.sync_copy(dst_hbm.at[pl.ds(e0, SEG_CHUNK)], dst_v)
        pltpu.async_copy(hs_hbm.at[src_v], rows_v, gsem).wait()
        pltpu.sync_copy(rows_v, acc.at[dst_v], add=True)

    plsc.subcore_barrier()
    row0 = s * ROWS_PER_TILE
    pltpu.sync_copy(acc.at[pl.ds(row0, ROWS_PER_TILE)],
                    out_hbm.at[c, pl.ds(row0, ROWS_PER_TILE)])


_BR = 1000
_GRID = N // _BR


def _prep_body(x_ref, w1_ref, d0_ref, d1_ref, hs_ref, dinvb_ref):
    deg = d0_ref[:, 0:1] + d1_ref[:, 0:1] + 1.0
    dinv = lax.rsqrt(deg)
    dinvb = jnp.broadcast_to(dinv, (_BR, D))
    h1 = jnp.dot(x_ref[...], w1_ref[...], preferred_element_type=jnp.float32)
    hs_ref[...] = h1 * dinvb
    dinvb_ref[...] = dinvb


def _mid_body(sa_ref, sb_ref, hs_ref, dinvb_ref, w2_ref, b1_ref,
              hs2_ref):
    dinvb = dinvb_ref[...]
    h = dinvb * (sa_ref[...] + sb_ref[...] + hs_ref[...]) + b1_ref[...]
    h = jnp.maximum(h, 0.0)
    h2 = jnp.dot(h, w2_ref[...], preferred_element_type=jnp.float32)
    hs2_ref[...] = h2 * dinvb


def _final_body(sa_ref, sb_ref, hs2_ref, dinvb_ref, b2_ref, out_ref):
    o = dinvb_ref[...] * (sa_ref[...] + sb_ref[...] + hs2_ref[...]) + b2_ref[...]
    m = jnp.max(o, axis=1, keepdims=True)
    z = o - m
    lse = jnp.log(jnp.sum(jnp.exp(z), axis=1, keepdims=True))
    out_ref[...] = z - lse


def _row_spec(w):
    return pl.BlockSpec((_BR, w), lambda i: (i, 0))


def _full_spec(h, w):
    return pl.BlockSpec((h, w), lambda i: (0, 0))


_prep = pl.pallas_call(
    _prep_body,
    grid=(_GRID,),
    in_specs=[_row_spec(D), _full_spec(D, D), _row_spec(HW), _row_spec(HW)],
    out_specs=[_row_spec(D), _row_spec(D)],
    out_shape=[jax.ShapeDtypeStruct((N, D), jnp.float32),
               jax.ShapeDtypeStruct((N, D), jnp.float32)],
)

_mid = pl.pallas_call(
    _mid_body,
    grid=(_GRID,),
    in_specs=[_row_spec(D), _row_spec(D), _row_spec(D), _row_spec(D),
              _full_spec(D, D), _full_spec(1, D)],
    out_specs=_row_spec(D),
    out_shape=jax.ShapeDtypeStruct((N, D), jnp.float32),
)

_final = pl.pallas_call(
    _final_body,
    grid=(_GRID,),
    in_specs=[_row_spec(D), _row_spec(D), _row_spec(D), _row_spec(D),
              _full_spec(1, D)],
    out_specs=_row_spec(D),
    out_shape=jax.ShapeDtypeStruct((N, D), jnp.float32),
)


@jax.jit
def kernel(x, edge_index, W1, b1, W2, b2):
    src = edge_index[0]
    dst = edge_index[1]

    degp = _sc_hist(dst)
    hs1, dinvb = _prep(x, W1, degp[0], degp[1])

    seg1 = _sc_seg(hs1, src, dst)
    hs2 = _mid(seg1[0], seg1[1], hs1, dinvb, W2, b1.reshape(1, D))

    seg2 = _sc_seg(hs2, src, dst)
    return _final(seg2[0], seg2[1], hs2, dinvb, b2.reshape(1, D))


# speedup vs baseline: 19.4810x; 19.4810x over previous
import functools

import jax
import jax.numpy as jnp
from jax import lax
from jax.experimental import pallas as pl
from jax.experimental.pallas import tpu as pltpu
from jax.experimental.pallas import tpu_sc as plsc

N = 10000
E = 320000
D = 128

NC = 2
NS = 16
E_PER_SC = E // NC
E_PER_W = E_PER_SC // NS
N_PAD = 10240
RPT = N_PAD // NS

SEG_CHUNK = 200
SEG_ITERS = E_PER_W // SEG_CHUNK

HIST_CHUNK = 2000
HIST_ITERS = E_PER_W // HIST_CHUNK
HW = 16

_sc_mesh = plsc.VectorSubcoreMesh(core_axis_name="c", subcore_axis_name="s")
_sc_params = pltpu.CompilerParams(use_tc_tiling_on_sc=False)


@functools.partial(
    pl.kernel,
    out_type=jax.ShapeDtypeStruct((NC, N_PAD, HW), jnp.float32),
    mesh=_sc_mesh,
    scratch_types=[
        pltpu.VMEM((HIST_CHUNK,), jnp.int32),
        pltpu.VMEM((HIST_CHUNK, HW), jnp.float32),
        pltpu.VMEM((RPT, HW), jnp.float32),
        pltpu.VMEM_SHARED((N_PAD, HW), jnp.float32),
    ],
    compiler_params=_sc_params,
)
def _sc_hist(dst_hbm, out_hbm, dst_v, ones_v, zero_v, acc):
    c = lax.axis_index("c")
    s = lax.axis_index("s")

    one16 = jnp.ones((16,), jnp.float32)
    zer16 = jnp.zeros((16,), jnp.float32)

    @pl.loop(0, HIST_CHUNK)
    def _(i):
        ones_v[i, :] = one16

    @pl.loop(0, RPT)
    def _(i):
        zero_v[i, :] = zer16

    pltpu.sync_copy(zero_v, acc.at[pl.ds(s * RPT, RPT)])
    plsc.subcore_barrier()

    base = c * E_PER_SC + s * E_PER_W

    @pl.loop(0, HIST_ITERS)
    def _(i):
        pltpu.sync_copy(dst_hbm.at[pl.ds(base + i * HIST_CHUNK, HIST_CHUNK)],
                        dst_v)
        pltpu.sync_copy(ones_v, acc.at[dst_v], add=True)

    plsc.subcore_barrier()
    row0 = s * RPT
    pltpu.sync_copy(acc.at[pl.ds(row0, RPT)],
                    out_hbm.at[c, pl.ds(row0, RPT)])


@functools.partial(
    pl.kernel,
    out_type=jax.ShapeDtypeStruct((NC, N_PAD, D), jnp.float32),
    mesh=_sc_mesh,
    scratch_types=[
        pltpu.VMEM((SEG_CHUNK,), jnp.int32),
        pltpu.VMEM((SEG_CHUNK,), jnp.int32),
        pltpu.VMEM((SEG_CHUNK, D), jnp.float32),
        pltpu.VMEM_SHARED((N_PAD, D), jnp.float32),
        pltpu.SemaphoreType.DMA,
    ],
    compiler_params=_sc_params,
)
def _sc_seg(hs_hbm, src_hbm, dst_hbm, out_hbm,
            src_v, dst_v, rows_v, acc, gsem):
    c = lax.axis_index("c")
    s = lax.axis_index("s")

    zer16 = jnp.zeros((16,), jnp.float32)
    zrows = RPT // 10

    @pl.loop(0, zrows)
    def _(i):
        for j in range(D // 16):
            rows_v[i, pl.ds(j * 16, 16)] = zer16

    for k in range(10):
        pltpu.sync_copy(
            rows_v.at[pl.ds(0, zrows)],
            acc.at[pl.ds(s * RPT + k * zrows, zrows)])
    plsc.subcore_barrier()

    base = c * E_PER_SC + s * E_PER_W

    @pl.loop(0, SEG_ITERS)
    def _(i):
        e0 = base + i * SEG_CHUNK
        pltpu.sync_copy(src_hbm.at[pl.ds(e0, SEG_CHUNK)], src_v)
        pltpu.sync_copy(dst_hbm.at[pl.ds(e0, SEG_CHUNK)], dst_v)
        pltpu.async_copy(hs_hbm.at[src_v], rows_v, gsem).wait()
        pltpu.sync_copy(rows_v, acc.at[dst_v], add=True)

    plsc.subcore_barrier()
    row0 = s * RPT
    pltpu.sync_copy(acc.at[pl.ds(row0, RPT)],
                    out_hbm.at[c, pl.ds(row0, RPT)])


_BR = 1000
_GRID = N // _BR


def _prep_body(x_ref, w1_ref, d0_ref, d1_ref, hs_ref, dinvb_ref):
    deg = d0_ref[:, 0:1] + d1_ref[:, 0:1] + 1.0
    dinv = lax.rsqrt(deg)
    dinvb = jnp.broadcast_to(dinv, (_BR, D))
    h1 = jnp.dot(x_ref[...], w1_ref[...], preferred_element_type=jnp.float32)
    hs_ref[...] = h1 * dinvb
    dinvb_ref[...] = dinvb


def _mid_body(sa_ref, sb_ref, hs_ref, dinvb_ref, w2_ref, b1_ref,
              hs2_ref):
    dinvb = dinvb_ref[...]
    h = dinvb * (sa_ref[...] + sb_ref[...] + hs_ref[...]) + b1_ref[...]
    h = jnp.maximum(h, 0.0)
    h2 = jnp.dot(h, w2_ref[...], preferred_element_type=jnp.float32)
    hs2_ref[...] = h2 * dinvb


def _final_body(sa_ref, sb_ref, hs2_ref, dinvb_ref, b2_ref, out_ref):
    o = dinvb_ref[...] * (sa_ref[...] + sb_ref[...] + hs2_ref[...]) + b2_ref[...]
    m = jnp.max(o, axis=1, keepdims=True)
    z = o - m
    lse = jnp.log(jnp.sum(jnp.exp(z), axis=1, keepdims=True))
    out_ref[...] = z - lse


def _row_spec(w):
    return pl.BlockSpec((_BR, w), lambda i: (i, 0))


def _full_spec(h, w):
    return pl.BlockSpec((h, w), lambda i: (0, 0))


_prep = pl.pallas_call(
    _prep_body,
    grid=(_GRID,),
    in_specs=[_row_spec(D), _full_spec(D, D), _row_spec(HW), _row_spec(HW)],
    out_specs=[_row_spec(D), _row_spec(D)],
    out_shape=[jax.ShapeDtypeStruct((N, D), jnp.float32),
               jax.ShapeDtypeStruct((N, D), jnp.float32)],
)

_mid = pl.pallas_call(
    _mid_body,
    grid=(_GRID,),
    in_specs=[_row_spec(D), _row_spec(D), _row_spec(D), _row_spec(D),
              _full_spec(D, D), _full_spec(1, D)],
    out_specs=_row_spec(D),
    out_shape=jax.ShapeDtypeStruct((N, D), jnp.float32),
)

_final = pl.pallas_call(
    _final_body,
    grid=(_GRID,),
    in_specs=[_row_spec(D), _row_spec(D), _row_spec(D), _row_spec(D),
              _full_spec(1, D)],
    out_specs=_row_spec(D),
    out_shape=jax.ShapeDtypeStruct((N, D), jnp.float32),
)


@jax.jit
def kernel(x, edge_index, W1, b1, W2, b2):
    src = edge_index[0]
    dst = edge_index[1]

    degp = _sc_hist(dst)
    hs1, dinvb = _prep(x, W1, degp[0], degp[1])

    seg1 = _sc_seg(hs1, src, dst)
    hs2 = _mid(seg1[0], seg1[1], hs1, dinvb, W2, b1.reshape(1, D))

    seg2 = _sc_seg(hs2, src, dst)
    return _final(seg2[0], seg2[1], hs2, dinvb, b2.reshape(1, D))


# speedup vs baseline: 24.1523x; 1.2398x over previous
import functools

import jax
import jax.numpy as jnp
from jax import lax
from jax.experimental import pallas as pl
from jax.experimental.pallas import tpu as pltpu
from jax.experimental.pallas import tpu_sc as plsc

N = 10000
E = 320000
D = 128

NC = 2
NS = 16
E_PER_SC = E // NC
E_PER_W = E_PER_SC // NS
N_PAD = 10240
RPT = N_PAD // NS

SEG_CHUNK = 100
SEG_ITERS = E_PER_W // SEG_CHUNK

HIST_CHUNK = 2000
HIST_ITERS = E_PER_W // HIST_CHUNK
HW = 16

_sc_mesh = plsc.VectorSubcoreMesh(core_axis_name="c", subcore_axis_name="s")
_sc_params = pltpu.CompilerParams(use_tc_tiling_on_sc=False)


@functools.partial(
    pl.kernel,
    out_type=jax.ShapeDtypeStruct((NC, N_PAD, HW), jnp.float32),
    mesh=_sc_mesh,
    scratch_types=[
        pltpu.VMEM((HIST_CHUNK,), jnp.int32),
        pltpu.VMEM((HIST_CHUNK, HW), jnp.float32),
        pltpu.VMEM((RPT, HW), jnp.float32),
        pltpu.VMEM_SHARED((N_PAD, HW), jnp.float32),
    ],
    compiler_params=_sc_params,
)
def _sc_hist(dst_hbm, out_hbm, dst_v, ones_v, zero_v, acc):
    c = lax.axis_index("c")
    s = lax.axis_index("s")

    one16 = jnp.ones((16,), jnp.float32)
    zer16 = jnp.zeros((16,), jnp.float32)

    @pl.loop(0, HIST_CHUNK)
    def _(i):
        ones_v[i, :] = one16

    @pl.loop(0, RPT)
    def _(i):
        zero_v[i, :] = zer16

    pltpu.sync_copy(zero_v, acc.at[pl.ds(s * RPT, RPT)])
    plsc.subcore_barrier()

    base = c * E_PER_SC + s * E_PER_W

    @pl.loop(0, HIST_ITERS)
    def _(i):
        pltpu.sync_copy(dst_hbm.at[pl.ds(base + i * HIST_CHUNK, HIST_CHUNK)],
                        dst_v)
        pltpu.sync_copy(ones_v, acc.at[dst_v], add=True)

    plsc.subcore_barrier()
    row0 = s * RPT
    pltpu.sync_copy(acc.at[pl.ds(row0, RPT)],
                    out_hbm.at[c, pl.ds(row0, RPT)])


@functools.partial(
    pl.kernel,
    out_type=jax.ShapeDtypeStruct((NC, N_PAD, D), jnp.float32),
    mesh=_sc_mesh,
    scratch_types=[
        pltpu.VMEM((SEG_ITERS, SEG_CHUNK), jnp.int32),
        pltpu.VMEM((SEG_ITERS, SEG_CHUNK), jnp.int32),
        pltpu.VMEM((SEG_CHUNK, D), jnp.float32),
        pltpu.VMEM((SEG_CHUNK, D), jnp.float32),
        pltpu.VMEM_SHARED((N_PAD, D), jnp.float32),
        pltpu.SemaphoreType.DMA,
        pltpu.SemaphoreType.DMA,
        pltpu.SemaphoreType.DMA,
        pltpu.SemaphoreType.DMA,
    ],
    compiler_params=_sc_params,
)
def _sc_seg(hs_hbm, src3_hbm, dst3_hbm, out_hbm,
            src_all, dst_all, rows0, rows1, acc, g0, g1, s0, s1):
    c = lax.axis_index("c")
    s = lax.axis_index("s")
    w = c * NS + s

    pltpu.sync_copy(src3_hbm.at[w], src_all)
    pltpu.sync_copy(dst3_hbm.at[w], dst_all)

    zer16 = jnp.zeros((16,), jnp.float32)
    zrows = RPT // 10

    @pl.loop(0, zrows)
    def _(i):
        for j in range(D // 16):
            rows0[i, pl.ds(j * 16, 16)] = zer16

    for k in range(10):
        pltpu.sync_copy(
            rows0.at[pl.ds(0, zrows)],
            acc.at[pl.ds(s * RPT + k * zrows, zrows)])
    plsc.subcore_barrier()

    rows = (rows0, rows1)
    gsem = (g0, g1)
    ssem = (s0, s1)

    def start_gather(i, b):
        pltpu.async_copy(hs_hbm.at[src_all.at[i]], rows[b], gsem[b])

    def wait_gather(b):
        pltpu.make_async_copy(hs_hbm.at[src_all.at[0]], rows[b],
                              gsem[b]).wait()

    def start_scatter(i, b):
        pltpu.async_copy(rows[b], acc.at[dst_all.at[i]], ssem[b], add=True)

    def wait_scatter(b):
        pltpu.make_async_copy(rows[b], acc.at[dst_all.at[0]],
                              ssem[b]).wait()

    start_gather(0, 0)
    start_gather(1, 1)
    wait_gather(0)
    start_scatter(0, 0)
    wait_gather(1)
    start_scatter(1, 1)

    @pl.loop(0, (SEG_ITERS - 2) // 2)
    def _(t):
        i = 2 * t + 2
        wait_scatter(0)
        start_gather(i, 0)
        wait_scatter(1)
        start_gather(i + 1, 1)
        wait_gather(0)
        start_scatter(i, 0)
        wait_gather(1)
        start_scatter(i + 1, 1)

    wait_scatter(0)
    wait_scatter(1)

    plsc.subcore_barrier()
    row0 = s * RPT
    pltpu.sync_copy(acc.at[pl.ds(row0, RPT)],
                    out_hbm.at[c, pl.ds(row0, RPT)])


_BR = 1000
_GRID = N // _BR


def _prep_body(x_ref, w1_ref, d0_ref, d1_ref, hs_ref, dinvb_ref):
    deg = d0_ref[:, 0:1] + d1_ref[:, 0:1] + 1.0
    dinv = lax.rsqrt(deg)
    dinvb = jnp.broadcast_to(dinv, (_BR, D))
    h1 = jnp.dot(x_ref[...], w1_ref[...], preferred_element_type=jnp.float32)
    hs_ref[...] = h1 * dinvb
    dinvb_ref[...] = dinvb


def _mid_body(sa_ref, sb_ref, hs_ref, dinvb_ref, w2_ref, b1_ref,
              hs2_ref):
    dinvb = dinvb_ref[...]
    h = dinvb * (sa_ref[...] + sb_ref[...] + hs_ref[...]) + b1_ref[...]
    h = jnp.maximum(h, 0.0)
    h2 = jnp.dot(h, w2_ref[...], preferred_element_type=jnp.float32)
    hs2_ref[...] = h2 * dinvb


def _final_body(sa_ref, sb_ref, hs2_ref, dinvb_ref, b2_ref, out_ref):
    o = dinvb_ref[...] * (sa_ref[...] + sb_ref[...] + hs2_ref[...]) + b2_ref[...]
    m = jnp.max(o, axis=1, keepdims=True)
    z = o - m
    lse = jnp.log(jnp.sum(jnp.exp(z), axis=1, keepdims=True))
    out_ref[...] = z - lse


def _row_spec(w):
    return pl.BlockSpec((_BR, w), lambda i: (i, 0))


def _full_spec(h, w):
    return pl.BlockSpec((h, w), lambda i: (0, 0))


_prep = pl.pallas_call(
    _prep_body,
    grid=(_GRID,),
    in_specs=[_row_spec(D), _full_spec(D, D), _row_spec(HW), _row_spec(HW)],
    out_specs=[_row_spec(D), _row_spec(D)],
    out_shape=[jax.ShapeDtypeStruct((N, D), jnp.float32),
               jax.ShapeDtypeStruct((N, D), jnp.float32)],
)

_mid = pl.pallas_call(
    _mid_body,
    grid=(_GRID,),
    in_specs=[_row_spec(D), _row_spec(D), _row_spec(D), _row_spec(D),
              _full_spec(D, D), _full_spec(1, D)],
    out_specs=_row_spec(D),
    out_shape=jax.ShapeDtypeStruct((N, D), jnp.float32),
)

_final = pl.pallas_call(
    _final_body,
    grid=(_GRID,),
    in_specs=[_row_spec(D), _row_spec(D), _row_spec(D), _row_spec(D),
              _full_spec(1, D)],
    out_specs=_row_spec(D),
    out_shape=jax.ShapeDtypeStruct((N, D), jnp.float32),
)


@jax.jit
def kernel(x, edge_index, W1, b1, W2, b2):
    src = edge_index[0]
    dst = edge_index[1]

    src3 = src.reshape(NC * NS, SEG_ITERS, SEG_CHUNK)
    dst3 = dst.reshape(NC * NS, SEG_ITERS, SEG_CHUNK)

    degp = _sc_hist(dst)
    hs1, dinvb = _prep(x, W1, degp[0], degp[1])

    seg1 = _sc_seg(hs1, src3, dst3)
    hs2 = _mid(seg1[0], seg1[1], hs1, dinvb, W2, b1.reshape(1, D))

    seg2 = _sc_seg(hs2, src3, dst3)
    return _final(seg2[0], seg2[1], hs2, dinvb, b2.reshape(1, D))
